# Initial kernel scaffold; baseline (speedup 1.0000x reference)
import jax
import jax.numpy as jnp
from jax.experimental import pallas as pl


def kernel(x, edge_index, edge_attr, W1, b1, W2, b2, W3, b3, W_fc3, b_fc3):
    raise NotImplementedError("write your pallas kernel here")


# speedup vs baseline: 5.6502x; 5.6502x over previous
import functools

import jax
import jax.numpy as jnp
from jax import lax
from jax.experimental import pallas as pl
from jax.experimental.pallas import tpu as pltpu
from jax.experimental.pallas import tpu_sc as plsc

F32 = jnp.float32
I32 = jnp.int32

NC = 2
NS = 16
LANES = 16
NW = NC * NS


def _mesh():
    return plsc.VectorSubcoreMesh(core_axis_name="c", subcore_axis_name="s")


_SC_PARAMS = pltpu.CompilerParams(needs_layout_passes=False,
                                  use_tc_tiling_on_sc=False)


def _lane_iota():
    return lax.iota(I32, LANES)


def _make_deg(E, NP):
    EW = E // NW
    K = 80
    NCHUNK = EW // K
    ZR = NP // NS

    @functools.partial(
        pl.kernel,
        mesh=_mesh(),
        out_type=jax.ShapeDtypeStruct((NC, NP, LANES), F32),
        compiler_params=_SC_PARAMS,
        scratch_types=[
            pltpu.VMEM_SHARED((NP, LANES), F32),
            pltpu.VMEM((ZR, LANES), F32),
            pltpu.VMEM((K,), I32),
            pltpu.VMEM((K,), F32),
            pltpu.VMEM((K, LANES), F32),
        ],
    )
    def deg_kernel(col_hbm, ea_hbm, out_hbm, acc, zbuf, col_v, ea_v, rep_v):
        c = lax.axis_index("c")
        s = lax.axis_index("s")
        zero16 = jnp.zeros((LANES,), F32)
        lane = _lane_iota()

        def zbody(i, carry):
            zbuf[i] = zero16
            return carry

        lax.fori_loop(0, ZR, zbody, 0)
        pltpu.sync_copy(zbuf, acc.at[pl.ds(ZR * s, ZR)])
        plsc.subcore_barrier()

        base0 = (c * NS + s) * EW

        def chunk(i, carry):
            base = base0 + i * K
            pltpu.sync_copy(col_hbm.at[pl.ds(base, K)], col_v)
            pltpu.sync_copy(ea_hbm.at[pl.ds(base, K)], ea_v)

            def gbody(g, cc):
                w16 = jnp.maximum(ea_v[pl.ds(g * LANES, LANES)], 0.0)
                for j in range(LANES):
                    nb = jnp.sum(jnp.where(lane == j, w16, 0.0))
                    rep_v[g * LANES + j] = jnp.full((LANES,), 0.0, F32) + nb
                return cc

            lax.fori_loop(0, K // LANES, gbody, 0)
            pltpu.sync_copy(rep_v, acc.at[col_v], add=True)
            return carry

        lax.fori_loop(0, NCHUNK, chunk, 0)
        plsc.subcore_barrier()
        pltpu.sync_copy(acc.at[pl.ds(ZR * s, ZR)],
                        out_hbm.at[c, pl.ds(ZR * s, ZR)])

    return deg_kernel


def _make_agg(E, NP, Dh):
    EW = E // NS
    K = 80
    NCH = EW // K
    RW = NP // NS
    JG = Dh // LANES

    @functools.partial(
        pl.kernel,
        mesh=_mesh(),
        out_type=(jax.ShapeDtypeStruct((NP, Dh), F32),
                  jax.ShapeDtypeStruct((NP, Dh), F32)),
        compiler_params=_SC_PARAMS,
        scratch_types=[
            pltpu.VMEM_SHARED((NP, Dh), F32),
            pltpu.VMEM((K,), I32),
            pltpu.VMEM((K,), I32),
            pltpu.VMEM((K,), F32),
            pltpu.VMEM((K, Dh), F32),
            pltpu.SemaphoreType.DMA,
        ],
    )
    def agg_kernel(row_hbm, col_hbm, ea_hbm, xq_lo, xq_hi, init_lo, init_hi,
                   out_lo, out_hi, acc, row_v, col_v, ea_v, rows_v, sem):
        c = lax.axis_index("c")
        s = lax.axis_index("s")
        lane = _lane_iota()

        def body(xq_h, init_h, out_h):
            pltpu.sync_copy(init_h.at[pl.ds(RW * s, RW)],
                            acc.at[pl.ds(RW * s, RW)])
            plsc.subcore_barrier()
            base0 = s * EW

            def chunk(i, carry):
                base = base0 + i * K
                pltpu.sync_copy(row_hbm.at[pl.ds(base, K)], row_v)
                pltpu.sync_copy(col_hbm.at[pl.ds(base, K)], col_v)
                pltpu.sync_copy(ea_hbm.at[pl.ds(base, K)], ea_v)
                pltpu.async_copy(xq_h.at[row_v], rows_v, sem).wait()

                def gbody(g, cc):
                    w16 = jnp.maximum(ea_v[pl.ds(g * LANES, LANES)], 0.0)
                    for j in range(LANES):
                        nb = jnp.sum(jnp.where(lane == j, w16, 0.0))
                        e = g * LANES + j
                        for jj in range(JG):
                            sl = pl.ds(jj * LANES, LANES)
                            rows_v[e, sl] = rows_v[e, sl] * nb
                    return cc

                lax.fori_loop(0, K // LANES, gbody, 0)
                pltpu.sync_copy(rows_v, acc.at[col_v], add=True)
                return carry

            lax.fori_loop(0, NCH, chunk, 0)
            plsc.subcore_barrier()
            pltpu.sync_copy(acc.at[pl.ds(RW * s, RW)],
                            out_h.at[pl.ds(RW * s, RW)])

        @pl.when(c == 0)
        def _():
            body(xq_lo, init_lo, out_lo)

        @pl.when(c == 1)
        def _():
            body(xq_hi, init_hi, out_hi)

    return agg_kernel


def _dinv_call(deg2d):
    def body(d_ref, dinv_ref, rdinv_ref):
        deg = d_ref[0] + d_ref[1] + 1.0
        dinv_ref[...] = lax.rsqrt(deg)
        rdinv_ref[...] = jnp.sqrt(deg)

    sh = deg2d.shape[1:]
    return pl.pallas_call(
        body,
        out_shape=(jax.ShapeDtypeStruct(sh, F32),
                   jax.ShapeDtypeStruct(sh, F32)),
    )(deg2d)


def _mm_first(x, W, b, dinv_c, rdinv_c, br=2048):
    N, DI = x.shape
    DO = W.shape[1]
    Dh = DO // 2

    def body(x_ref, w_ref, b_ref, di_ref, rd_ref, xlo, xhi, ilo, ihi):
        xw = jnp.dot(x_ref[...], w_ref[...], preferred_element_type=F32)
        xq = xw * di_ref[...]
        init = xq + b_ref[...] * rd_ref[...]
        xlo[...] = xq[:, :Dh]
        xhi[...] = xq[:, Dh:]
        ilo[...] = init[:, :Dh]
        ihi[...] = init[:, Dh:]

    outs = tuple(jax.ShapeDtypeStruct((N, Dh), F32) for _ in range(4))
    bo = pl.BlockSpec((br, Dh), lambda i: (i, 0))
    bc = pl.BlockSpec((br, 1), lambda i: (i, 0))
    return pl.pallas_call(
        body,
        grid=(N // br,),
        in_specs=[pl.BlockSpec((br, DI), lambda i: (i, 0)),
                  pl.BlockSpec((DI, DO), lambda i: (0, 0)),
                  pl.BlockSpec((1, DO), lambda i: (0, 0)),
                  bc, bc],
        out_specs=[bo, bo, bo, bo],
        out_shape=outs,
    )(x, W, b.reshape(1, DO), dinv_c, rdinv_c)


def _mm_mid(slo, shi, W, b, dinv_c, rdinv_c, br=2048):
    N, Dhin = slo.shape
    DI, DO = W.shape
    Dh = DO // 2

    def body(lo_ref, hi_ref, w_ref, b_ref, di_ref, rd_ref,
             xlo, xhi, ilo, ihi):
        di = di_ref[...]
        hlo = jnp.maximum(lo_ref[...] * di, 0.0)
        hhi = jnp.maximum(hi_ref[...] * di, 0.0)
        w = w_ref[...]
        xw = (jnp.dot(hlo, w[:Dhin], preferred_element_type=F32)
              + jnp.dot(hhi, w[Dhin:], preferred_element_type=F32))
        xq = xw * di
        init = xq + b_ref[...] * rd_ref[...]
        xlo[...] = xq[:, :Dh]
        xhi[...] = xq[:, Dh:]
        ilo[...] = init[:, :Dh]
        ihi[...] = init[:, Dh:]

    outs = tuple(jax.ShapeDtypeStruct((N, Dh), F32) for _ in range(4))
    bi = pl.BlockSpec((br, Dhin), lambda i: (i, 0))
    bo = pl.BlockSpec((br, Dh), lambda i: (i, 0))
    bc = pl.BlockSpec((br, 1), lambda i: (i, 0))
    return pl.pallas_call(
        body,
        grid=(N // br,),
        in_specs=[bi, bi,
                  pl.BlockSpec((DI, DO), lambda i: (0, 0)),
                  pl.BlockSpec((1, DO), lambda i: (0, 0)),
                  bc, bc],
        out_specs=[bo, bo, bo, bo],
        out_shape=outs,
    )(slo, shi, W, b.reshape(1, DO), dinv_c, rdinv_c)


def _head(slo, shi, W, b, dinv_c, br=2048):
    N, Dhin = slo.shape
    DI, DO = W.shape

    def body(lo_ref, hi_ref, w_ref, b_ref, di_ref, o_ref):
        di = di_ref[...]
        hlo = jnp.maximum(lo_ref[...] * di, 0.0)
        hhi = jnp.maximum(hi_ref[...] * di, 0.0)
        w = w_ref[...]
        out = (jnp.dot(hlo, w[:Dhin], preferred_element_type=F32)
               + jnp.dot(hhi, w[Dhin:], preferred_element_type=F32))
        out = out + b_ref[...]
        o_ref[...] = jnp.where(out > 0, out, 0.2 * out)

    bi = pl.BlockSpec((br, Dhin), lambda i: (i, 0))
    bc = pl.BlockSpec((br, 1), lambda i: (i, 0))
    return pl.pallas_call(
        body,
        grid=(N // br,),
        in_specs=[bi, bi,
                  pl.BlockSpec((DI, DO), lambda i: (0, 0)),
                  pl.BlockSpec((1, DO), lambda i: (0, 0)),
                  bc],
        out_specs=pl.BlockSpec((br, DO), lambda i: (i, 0)),
        out_shape=jax.ShapeDtypeStruct((N, DO), F32),
    )(slo, shi, W, b.reshape(1, DO), dinv_c)


def kernel(x, edge_index, edge_attr, W1, b1, W2, b2, W3, b3, W_fc3, b_fc3):
    N, DI = x.shape
    E = edge_index.shape[1]
    NP = ((N + 2047) // 2048) * 2048

    row = edge_index[0]
    col = edge_index[1]
    xp = jnp.pad(x, ((0, NP - N), (0, 0)))

    deg_part = _make_deg(E, NP)(col, edge_attr)
    deg2d = deg_part[:, :, 0].reshape(NC, NP // 128, 128)
    dinv, rdinv = _dinv_call(deg2d)
    dinv_c = dinv.reshape(NP, 1)
    rdinv_c = rdinv.reshape(NP, 1)

    xq_lo, xq_hi, i_lo, i_hi = _mm_first(xp, W1, b1, dinv_c, rdinv_c)
    s_lo, s_hi = _make_agg(E, NP, W1.shape[1] // 2)(
        row, col, edge_attr, xq_lo, xq_hi, i_lo, i_hi)

    xq_lo, xq_hi, i_lo, i_hi = _mm_mid(s_lo, s_hi, W2, b2, dinv_c, rdinv_c)
    s_lo, s_hi = _make_agg(E, NP, W2.shape[1] // 2)(
        row, col, edge_attr, xq_lo, xq_hi, i_lo, i_hi)

    xq_lo, xq_hi, i_lo, i_hi = _mm_mid(s_lo, s_hi, W3, b3, dinv_c, rdinv_c)
    s_lo, s_hi = _make_agg(E, NP, W3.shape[1] // 2)(
        row, col, edge_attr, xq_lo, xq_hi, i_lo, i_hi)

    return _head(s_lo, s_hi, W_fc3, b_fc3, dinv_c)[:N]


# speedup vs baseline: 13.3205x; 2.3575x over previous
import functools

import jax
import jax.numpy as jnp
from jax import lax
from jax.experimental import pallas as pl
from jax.experimental.pallas import tpu as pltpu
from jax.experimental.pallas import tpu_sc as plsc

F32 = jnp.float32
I32 = jnp.int32

NC = 2
NS = 16
LANES = 16
NW = NC * NS


def _mesh():
    return plsc.VectorSubcoreMesh(core_axis_name="c", subcore_axis_name="s")


_SC_PARAMS = pltpu.CompilerParams(needs_layout_passes=False,
                                  use_tc_tiling_on_sc=False)


def _lane_iota():
    return lax.iota(I32, LANES)


def _make_deg(E, NP):
    EW = E // NW
    K = 80
    NCHUNK = EW // K
    ZR = NP // NS

    @functools.partial(
        pl.kernel,
        mesh=_mesh(),
        out_type=jax.ShapeDtypeStruct((NC, NP, LANES), F32),
        compiler_params=_SC_PARAMS,
        scratch_types=[
            pltpu.VMEM_SHARED((NP, LANES), F32),
            pltpu.VMEM((ZR, LANES), F32),
            pltpu.VMEM((K,), I32),
            pltpu.VMEM((K,), F32),
            pltpu.VMEM((K, LANES), F32),
        ],
    )
    def deg_kernel(col_hbm, ea_hbm, out_hbm, acc, zbuf, col_v, ea_v, rep_v):
        c = lax.axis_index("c")
        s = lax.axis_index("s")
        zero16 = jnp.zeros((LANES,), F32)
        lane = _lane_iota()

        def zbody(i, carry):
            zbuf[i] = zero16
            return carry

        lax.fori_loop(0, ZR, zbody, 0)
        pltpu.sync_copy(zbuf, acc.at[pl.ds(ZR * s, ZR)])
        plsc.subcore_barrier()

        base0 = (c * NS + s) * EW

        def chunk(i, carry):
            base = base0 + i * K
            pltpu.sync_copy(col_hbm.at[pl.ds(base, K)], col_v)
            pltpu.sync_copy(ea_hbm.at[pl.ds(base, K)], ea_v)

            def gbody(g, cc):
                w16 = jnp.maximum(ea_v[pl.ds(g * LANES, LANES)], 0.0)
                for j in range(LANES):
                    nb = jnp.sum(jnp.where(lane == j, w16, 0.0))
                    rep_v[g * LANES + j] = jnp.full((LANES,), 0.0, F32) + nb
                return cc

            lax.fori_loop(0, K // LANES, gbody, 0)
            pltpu.sync_copy(rep_v, acc.at[col_v], add=True)
            return carry

        lax.fori_loop(0, NCHUNK, chunk, 0)
        plsc.subcore_barrier()
        pltpu.sync_copy(acc.at[pl.ds(ZR * s, ZR)],
                        out_hbm.at[c, pl.ds(ZR * s, ZR)])

    return deg_kernel


def _make_agg(E, NP, Dh):
    EW = E // NS
    K = 80
    NCH = EW // K
    RW = NP // NS
    JG = Dh // LANES

    NPAIR = NCH // 2
    GPC = K // LANES

    @functools.partial(
        pl.kernel,
        mesh=_mesh(),
        out_type=(jax.ShapeDtypeStruct((NP, Dh), F32),
                  jax.ShapeDtypeStruct((NP, Dh), F32)),
        compiler_params=_SC_PARAMS,
        scratch_types=[
            pltpu.VMEM_SHARED((NP, Dh), F32),
            pltpu.VMEM((K,), I32), pltpu.VMEM((K,), I32),
            pltpu.VMEM((K,), F32), pltpu.VMEM((K,), I32),
            pltpu.VMEM((K, Dh), F32),
            pltpu.VMEM((K,), I32), pltpu.VMEM((K,), I32),
            pltpu.VMEM((K,), F32), pltpu.VMEM((K,), I32),
            pltpu.VMEM((K, Dh), F32),
            pltpu.SemaphoreType.DMA, pltpu.SemaphoreType.DMA,
            pltpu.SemaphoreType.DMA, pltpu.SemaphoreType.DMA,
            pltpu.SemaphoreType.DMA, pltpu.SemaphoreType.DMA,
        ],
    )
    def agg_kernel(row_hbm, col_hbm, ea_hbm, xq_lo, xq_hi, init_lo, init_hi,
                   out_lo, out_hi, acc,
                   row_a, col_a, ea_a, scol_a, rows_a,
                   row_b, col_b, ea_b, scol_b, rows_b,
                   si_a, si_b, sg_a, sg_b, ss_a, ss_b):
        c = lax.axis_index("c")
        s = lax.axis_index("s")
        lane = _lane_iota()

        def body(xq_h, init_h, out_h):
            base0 = s * EW

            def issue_idx(i, row_v, col_v, ea_v, sem):
                base = base0 + i * K
                pltpu.async_copy(row_hbm.at[pl.ds(base, K)], row_v, sem)
                pltpu.async_copy(col_hbm.at[pl.ds(base, K)], col_v, sem)
                pltpu.async_copy(ea_hbm.at[pl.ds(base, K)], ea_v, sem)

            def wait_idx(row_v, col_v, ea_v, sem):
                pltpu.make_async_copy(row_hbm.at[pl.ds(0, K)], row_v,
                                      sem).wait()
                pltpu.make_async_copy(col_hbm.at[pl.ds(0, K)], col_v,
                                      sem).wait()
                pltpu.make_async_copy(ea_hbm.at[pl.ds(0, K)], ea_v,
                                      sem).wait()

            def wait_scatter(rows_v, scol_v, sem):
                pltpu.make_async_copy(rows_v, acc.at[scol_v], sem).wait()

            def scale(ea_v, rows_v):
                def gbody(g, cc):
                    w16 = jnp.maximum(ea_v[pl.ds(g * LANES, LANES)], 0.0)
                    for j in range(LANES):
                        nb = jnp.sum(jnp.where(lane == j, w16, 0.0))
                        e = g * LANES + j
                        for jj in range(JG):
                            sl = pl.ds(jj * LANES, LANES)
                            rows_v[e, sl] = rows_v[e, sl] * nb
                    return cc

                lax.fori_loop(0, GPC, gbody, 0)

            def copy_col(col_v, scol_v):
                for q in range(GPC):
                    sl = pl.ds(q * LANES, LANES)
                    scol_v[sl] = col_v[sl]

            issue_idx(0, row_a, col_a, ea_a, si_a)
            issue_idx(1, row_b, col_b, ea_b, si_b)
            pltpu.sync_copy(init_h.at[pl.ds(RW * s, RW)],
                            acc.at[pl.ds(RW * s, RW)])
            plsc.subcore_barrier()

            def pair(p, carry):
                wait_idx(row_a, col_a, ea_a, si_a)

                @pl.when(p > 0)
                def _():
                    wait_scatter(rows_a, scol_a, ss_a)

                pltpu.async_copy(xq_h.at[row_a], rows_a, sg_a)

                @pl.when(p > 0)
                def _():
                    wait_scatter(rows_b, scol_b, ss_b)

                wait_idx(row_b, col_b, ea_b, si_b)
                pltpu.async_copy(xq_h.at[row_b], rows_b, sg_b)

                pltpu.make_async_copy(xq_h.at[row_a], rows_a, sg_a).wait()
                copy_col(col_a, scol_a)
                scale(ea_a, rows_a)
                pltpu.async_copy(rows_a, acc.at[scol_a], ss_a, add=True)

                @pl.when(p < NPAIR - 1)
                def _():
                    issue_idx(2 * p + 2, row_a, col_a, ea_a, si_a)

                pltpu.make_async_copy(xq_h.at[row_b], rows_b, sg_b).wait()
                copy_col(col_b, scol_b)
                scale(ea_b, rows_b)
                pltpu.async_copy(rows_b, acc.at[scol_b], ss_b, add=True)

                @pl.when(p < NPAIR - 1)
                def _():
                    issue_idx(2 * p + 3, row_b, col_b, ea_b, si_b)

                return carry

            lax.fori_loop(0, NPAIR, pair, 0)
            wait_scatter(rows_a, scol_a, ss_a)
            wait_scatter(rows_b, scol_b, ss_b)
            plsc.subcore_barrier()
            pltpu.sync_copy(acc.at[pl.ds(RW * s, RW)],
                            out_h.at[pl.ds(RW * s, RW)])

        @pl.when(c == 0)
        def _():
            body(xq_lo, init_lo, out_lo)

        @pl.when(c == 1)
        def _():
            body(xq_hi, init_hi, out_hi)

    return agg_kernel


def _dinv_call(deg2d):
    def body(d_ref, dinv_ref, rdinv_ref):
        deg = d_ref[0] + d_ref[1] + 1.0
        dinv_ref[...] = lax.rsqrt(deg)
        rdinv_ref[...] = jnp.sqrt(deg)

    sh = deg2d.shape[1:]
    return pl.pallas_call(
        body,
        out_shape=(jax.ShapeDtypeStruct(sh, F32),
                   jax.ShapeDtypeStruct(sh, F32)),
    )(deg2d)


def _mm_first(x, W, b, dinv_c, rdinv_c, br=2048):
    N, DI = x.shape
    DO = W.shape[1]
    Dh = DO // 2

    def body(x_ref, w_ref, b_ref, di_ref, rd_ref, xlo, xhi, ilo, ihi):
        xw = jnp.dot(x_ref[...], w_ref[...], preferred_element_type=F32)
        xq = xw * di_ref[...]
        init = xq + b_ref[...] * rd_ref[...]
        xlo[...] = xq[:, :Dh]
        xhi[...] = xq[:, Dh:]
        ilo[...] = init[:, :Dh]
        ihi[...] = init[:, Dh:]

    outs = tuple(jax.ShapeDtypeStruct((N, Dh), F32) for _ in range(4))
    bo = pl.BlockSpec((br, Dh), lambda i: (i, 0))
    bc = pl.BlockSpec((br, 1), lambda i: (i, 0))
    return pl.pallas_call(
        body,
        grid=(N // br,),
        in_specs=[pl.BlockSpec((br, DI), lambda i: (i, 0)),
                  pl.BlockSpec((DI, DO), lambda i: (0, 0)),
                  pl.BlockSpec((1, DO), lambda i: (0, 0)),
                  bc, bc],
        out_specs=[bo, bo, bo, bo],
        out_shape=outs,
    )(x, W, b.reshape(1, DO), dinv_c, rdinv_c)


def _mm_mid(slo, shi, W, b, dinv_c, rdinv_c, br=2048):
    N, Dhin = slo.shape
    DI, DO = W.shape
    Dh = DO // 2

    def body(lo_ref, hi_ref, w_ref, b_ref, di_ref, rd_ref,
             xlo, xhi, ilo, ihi):
        di = di_ref[...]
        hlo = jnp.maximum(lo_ref[...] * di, 0.0)
        hhi = jnp.maximum(hi_ref[...] * di, 0.0)
        w = w_ref[...]
        xw = (jnp.dot(hlo, w[:Dhin], preferred_element_type=F32)
              + jnp.dot(hhi, w[Dhin:], preferred_element_type=F32))
        xq = xw * di
        init = xq + b_ref[...] * rd_ref[...]
        xlo[...] = xq[:, :Dh]
        xhi[...] = xq[:, Dh:]
        ilo[...] = init[:, :Dh]
        ihi[...] = init[:, Dh:]

    outs = tuple(jax.ShapeDtypeStruct((N, Dh), F32) for _ in range(4))
    bi = pl.BlockSpec((br, Dhin), lambda i: (i, 0))
    bo = pl.BlockSpec((br, Dh), lambda i: (i, 0))
    bc = pl.BlockSpec((br, 1), lambda i: (i, 0))
    return pl.pallas_call(
        body,
        grid=(N // br,),
        in_specs=[bi, bi,
                  pl.BlockSpec((DI, DO), lambda i: (0, 0)),
                  pl.BlockSpec((1, DO), lambda i: (0, 0)),
                  bc, bc],
        out_specs=[bo, bo, bo, bo],
        out_shape=outs,
    )(slo, shi, W, b.reshape(1, DO), dinv_c, rdinv_c)


def _head(slo, shi, W, b, dinv_c, br=2048):
    N, Dhin = slo.shape
    DI, DO = W.shape

    def body(lo_ref, hi_ref, w_ref, b_ref, di_ref, o_ref):
        di = di_ref[...]
        hlo = jnp.maximum(lo_ref[...] * di, 0.0)
        hhi = jnp.maximum(hi_ref[...] * di, 0.0)
        w = w_ref[...]
        out = (jnp.dot(hlo, w[:Dhin], preferred_element_type=F32)
               + jnp.dot(hhi, w[Dhin:], preferred_element_type=F32))
        out = out + b_ref[...]
        o_ref[...] = jnp.where(out > 0, out, 0.2 * out)

    bi = pl.BlockSpec((br, Dhin), lambda i: (i, 0))
    bc = pl.BlockSpec((br, 1), lambda i: (i, 0))
    return pl.pallas_call(
        body,
        grid=(N // br,),
        in_specs=[bi, bi,
                  pl.BlockSpec((DI, DO), lambda i: (0, 0)),
                  pl.BlockSpec((1, DO), lambda i: (0, 0)),
                  bc],
        out_specs=pl.BlockSpec((br, DO), lambda i: (i, 0)),
        out_shape=jax.ShapeDtypeStruct((N, DO), F32),
    )(slo, shi, W, b.reshape(1, DO), dinv_c)


def kernel(x, edge_index, edge_attr, W1, b1, W2, b2, W3, b3, W_fc3, b_fc3):
    N, DI = x.shape
    E = edge_index.shape[1]
    NP = ((N + 2047) // 2048) * 2048

    row = edge_index[0]
    col = edge_index[1]
    xp = jnp.pad(x, ((0, NP - N), (0, 0)))

    deg_part = _make_deg(E, NP)(col, edge_attr)
    deg2d = deg_part[:, :, 0].reshape(NC, NP // 128, 128)
    dinv, rdinv = _dinv_call(deg2d)
    dinv_c = dinv.reshape(NP, 1)
    rdinv_c = rdinv.reshape(NP, 1)

    xq_lo, xq_hi, i_lo, i_hi = _mm_first(xp, W1, b1, dinv_c, rdinv_c)
    s_lo, s_hi = _make_agg(E, NP, W1.shape[1] // 2)(
        row, col, edge_attr, xq_lo, xq_hi, i_lo, i_hi)

    xq_lo, xq_hi, i_lo, i_hi = _mm_mid(s_lo, s_hi, W2, b2, dinv_c, rdinv_c)
    s_lo, s_hi = _make_agg(E, NP, W2.shape[1] // 2)(
        row, col, edge_attr, xq_lo, xq_hi, i_lo, i_hi)

    xq_lo, xq_hi, i_lo, i_hi = _mm_mid(s_lo, s_hi, W3, b3, dinv_c, rdinv_c)
    s_lo, s_hi = _make_agg(E, NP, W3.shape[1] // 2)(
        row, col, edge_attr, xq_lo, xq_hi, i_lo, i_hi)

    return _head(s_lo, s_hi, W_fc3, b_fc3, dinv_c)[:N]
